# baseline (speedup 1.0000x reference)
import math

import jax
import jax.numpy as jnp
from jax import lax
from jax.experimental import pallas as pl
from jax.experimental.pallas import tpu as pltpu
from jax.experimental.pallas import tpu_sc as plsc
from jax._src.pallas import mpmd

N = 10000
E = 320000
HID = 128
HEADS = 8
CH = 16
NG = 16

NC, NS = 2, 16
NW = NC * NS
EPT = E // NW
NPAD = 10240
RPT = NPAD // NS

BNS = float(1.0 / math.sqrt(1.0 + 1e-5))

_VMESH = plsc.VectorSubcoreMesh(core_axis_name="c", subcore_axis_name="s",
                                num_cores=NC, num_subcores=NS)
_SMESH = plsc.ScalarSubcoreMesh(axis_name="c", num_cores=NC)


def _sc_map(tec_fn, out_types, shared_scratch):
    def scs_fn(*refs):
        pass
    return mpmd.mpmd_map(
        [(_SMESH, scs_fn), (_VMESH, tec_fn)],
        out_types=out_types,
        scratch_types=[shared_scratch],
        compiler_params=pltpu.CompilerParams(use_tc_tiling_on_sc=False, needs_layout_passes=False, internal_scratch_in_bytes=128 * 1024),
    )


CNT_C = 200


def _count_tec(col_hbm, onepat_hbm, zeros_hbm, out_hbm, acc):
    def inner(colv, srcv, sem):
        c = lax.axis_index("c")
        s = lax.axis_index("s")
        wid = s * NC + c
        pltpu.sync_copy(zeros_hbm.at[pl.ds(s * RPT, RPT)],
                        acc.at[pl.ds(s * RPT, RPT)])
        pltpu.sync_copy(onepat_hbm, srcv)
        plsc.subcore_barrier()

        def body(i, carry):
            base = wid * EPT + i * CNT_C
            pltpu.sync_copy(col_hbm.at[pl.ds(base, CNT_C)], colv)
            pltpu.sync_copy(srcv, acc.at[colv], add=True)
            return carry

        lax.fori_loop(0, EPT // CNT_C, body, 0)
        plsc.subcore_barrier()
        pltpu.sync_copy(acc.at[pl.ds(s * RPT, RPT)],
                        out_hbm.at[c, pl.ds(s * RPT, RPT)])

    pl.run_scoped(inner,
                  pltpu.VMEM((CNT_C,), jnp.int32),
                  pltpu.VMEM((CNT_C, 16), jnp.float32),
                  pltpu.SemaphoreType.DMA)


def _sc_count(col, onepat, zeros16):
    return _sc_map(
        _count_tec,
        jax.ShapeDtypeStruct((NC, NPAD, 16), jnp.float32),
        pltpu.VMEM_SHARED((NPAD, 16), jnp.float32),
    )(col, onepat, zeros16)


SEG_C = 184
SEG_NCH = EPT // SEG_C
SEG_TAIL = EPT - SEG_NCH * SEG_C


def _seg_tec(val_hbm, row_hbm, col_hbm, zeros_hbm, out_hbm, acc):
    def inner(rowv0, colv0, buf0, sem0, rowv1, colv1, buf1, sem1, trow, tcol,
              ssem0, ssem1):
        c = lax.axis_index("c")
        s = lax.axis_index("s")
        wid = s * NC + c
        rows = (rowv0, rowv1)
        cols = (colv0, colv1)
        bufs = (buf0, buf1)
        sems = (sem0, sem1)
        ssems = (ssem0, ssem1)
        pltpu.sync_copy(zeros_hbm.at[pl.ds(s * RPT, RPT)],
                        acc.at[pl.ds(s * RPT, RPT)])
        plsc.subcore_barrier()

        tbase = wid * EPT + SEG_NCH * SEG_C
        pltpu.sync_copy(row_hbm.at[pl.ds(tbase, SEG_TAIL)], trow)
        pltpu.sync_copy(col_hbm.at[pl.ds(tbase, SEG_TAIL)], tcol)
        pltpu.async_copy(val_hbm.at[trow], bufs[0].at[pl.ds(0, SEG_TAIL)],
                         sems[0]).wait()
        pltpu.sync_copy(bufs[0].at[pl.ds(0, SEG_TAIL)], acc.at[tcol], add=True)

        def start(i, k, wait_scatter):
            if wait_scatter:
                pltpu.make_async_copy(bufs[k], acc.at[cols[k]], ssems[k]).wait()
            base = wid * EPT + i * SEG_C
            pltpu.sync_copy(row_hbm.at[pl.ds(base, SEG_C)], rows[k])
            pltpu.sync_copy(col_hbm.at[pl.ds(base, SEG_C)], cols[k])
            return pltpu.async_copy(val_hbm.at[rows[k]], bufs[k], sems[k])

        cp = start(0, 0, False)
        for i in range(SEG_NCH):
            k = i % 2
            nxt = None
            if i + 1 < SEG_NCH:
                nxt = start(i + 1, 1 - k, i + 1 >= 2)
            cp.wait()
            pltpu.async_copy(bufs[k], acc.at[cols[k]], ssems[k], add=True)
            cp = nxt
        for k in (0, 1):
            pltpu.make_async_copy(bufs[k], acc.at[cols[k]], ssems[k]).wait()
        plsc.subcore_barrier()
        pltpu.sync_copy(acc.at[pl.ds(s * RPT, RPT)],
                        out_hbm.at[c, pl.ds(s * RPT, RPT)])

    pl.run_scoped(inner,
                  pltpu.VMEM((SEG_C,), jnp.int32),
                  pltpu.VMEM((SEG_C,), jnp.int32),
                  pltpu.VMEM((SEG_C, HID), jnp.float32),
                  pltpu.SemaphoreType.DMA,
                  pltpu.VMEM((SEG_C,), jnp.int32),
                  pltpu.VMEM((SEG_C,), jnp.int32),
                  pltpu.VMEM((SEG_C, HID), jnp.float32),
                  pltpu.SemaphoreType.DMA,
                  pltpu.VMEM((SEG_TAIL,), jnp.int32),
                  pltpu.VMEM((SEG_TAIL,), jnp.int32),
                  pltpu.SemaphoreType.DMA,
                  pltpu.SemaphoreType.DMA)


def _sc_segsum(val, row, col, zeros128):
    return _sc_map(
        _seg_tec,
        jax.ShapeDtypeStruct((NC, NPAD, HID), jnp.float32),
        pltpu.VMEM_SHARED((NPAD, HID), jnp.float32),
    )(val, row, col, zeros128)


A_C = 200
A_NCH = EPT // A_C


def _gata_tec(a16_hbm, b16_hbm, row_hbm, col_hbm, m_hbm, zeros_hbm,
              w_hbm, sout_hbm, acc):
    def inner(rowv0, colv0, abuf0, bbuf0, wbuf0, sem0,
              rowv1, colv1, abuf1, bbuf1, wbuf1, sem1, mv):
        c = lax.axis_index("c")
        s = lax.axis_index("s")
        wid = s * NC + c
        rows = (rowv0, rowv1)
        cols = (colv0, colv1)
        abufs = (abuf0, abuf1)
        bbufs = (bbuf0, bbuf1)
        wbufs = (wbuf0, wbuf1)
        sems = (sem0, sem1)
        pltpu.sync_copy(zeros_hbm.at[pl.ds(s * RPT, RPT)],
                        acc.at[pl.ds(s * RPT, RPT)])
        pltpu.sync_copy(m_hbm, mv)
        plsc.subcore_barrier()
        mvec = mv[...]

        def start(i, k):
            base = wid * EPT + i * A_C
            pltpu.sync_copy(row_hbm.at[pl.ds(base, A_C)], rows[k])
            pltpu.sync_copy(col_hbm.at[pl.ds(base, A_C)], cols[k])
            cpa = pltpu.async_copy(a16_hbm.at[rows[k]], abufs[k], sems[k])
            cpb = pltpu.async_copy(b16_hbm.at[cols[k]], bbufs[k], sems[k])
            return (cpa, cpb)

        cp = start(0, 0)
        for i in range(A_NCH):
            k = i % 2
            nxt = None
            if i + 1 < A_NCH:
                nxt = start(i + 1, 1 - k)
            cp[0].wait()
            cp[1].wait()

            def ebody(e, carry2):
                x = abufs[k][e] + bbufs[k][e]
                l = jnp.where(x > 0, x, 0.2 * x)
                wbufs[k][e] = jnp.exp(l - mvec)
                return carry2

            lax.fori_loop(0, A_C, ebody, 0, unroll=4)
            base = wid * EPT + i * A_C
            pltpu.sync_copy(wbufs[k], w_hbm.at[pl.ds(base, A_C)])
            pltpu.sync_copy(wbufs[k], acc.at[cols[k]], add=True)
            cp = nxt
        plsc.subcore_barrier()
        pltpu.sync_copy(acc.at[pl.ds(s * RPT, RPT)],
                        sout_hbm.at[c, pl.ds(s * RPT, RPT)])

    pl.run_scoped(inner,
                  pltpu.VMEM((A_C,), jnp.int32),
                  pltpu.VMEM((A_C,), jnp.int32),
                  pltpu.VMEM((A_C, 16), jnp.float32),
                  pltpu.VMEM((A_C, 16), jnp.float32),
                  pltpu.VMEM((A_C, 16), jnp.float32),
                  pltpu.SemaphoreType.DMA,
                  pltpu.VMEM((A_C,), jnp.int32),
                  pltpu.VMEM((A_C,), jnp.int32),
                  pltpu.VMEM((A_C, 16), jnp.float32),
                  pltpu.VMEM((A_C, 16), jnp.float32),
                  pltpu.VMEM((A_C, 16), jnp.float32),
                  pltpu.SemaphoreType.DMA,
                  pltpu.VMEM((16,), jnp.float32))


def _sc_gata(a16, b16, row, col, m16, zeros16):
    return _sc_map(
        _gata_tec,
        (jax.ShapeDtypeStruct((E, 16), jnp.float32),
         jax.ShapeDtypeStruct((NC, NPAD, 16), jnp.float32)),
        pltpu.VMEM_SHARED((NPAD, 16), jnp.float32),
    )(a16, b16, row, col, m16, zeros16)


B_C = 144
B_NCH = EPT // B_C
B_TAIL = EPT - B_NCH * B_C


def _gatb_tec(xw_hbm, w_hbm, row_hbm, col_hbm, zeros_hbm, out_hbm, acc):
    def inner(rowv0, colv0, wbuf0, gbuf0, sem0,
              rowv1, colv1, wbuf1, gbuf1, sem1, trow, tcol, ssem0, ssem1):
        ssems = (ssem0, ssem1)
        c = lax.axis_index("c")
        s = lax.axis_index("s")
        wid = s * NC + c
        rows = (rowv0, rowv1)
        cols = (colv0, colv1)
        wbufs = (wbuf0, wbuf1)
        gbufs = (gbuf0, gbuf1)
        sems = (sem0, sem1)
        pltpu.sync_copy(zeros_hbm.at[pl.ds(s * RPT, RPT)],
                        acc.at[pl.ds(s * RPT, RPT)])
        plsc.subcore_barrier()

        def mul(k, n):
            hidx = [jnp.full((16, 1), h, jnp.int32) for h in range(HEADS)]
            dn = lax.GatherDimensionNumbers(
                offset_dims=(), collapsed_slice_dims=(0,),
                start_index_map=(0,))

            def ebody(e, carry2):
                wrow = wbufs[k][e]
                for h in range(HEADS):
                    wv = lax.gather(
                        wrow, hidx[h], dn, (1,),
                        mode=lax.GatherScatterMode.PROMISE_IN_BOUNDS)
                    g = gbufs[k][e, pl.ds(h * 16, 16)]
                    gbufs[k][e, pl.ds(h * 16, 16)] = g * wv
                return carry2
            lax.fori_loop(0, n, ebody, 0, unroll=4)

        tbase = wid * EPT + B_NCH * B_C
        pltpu.sync_copy(row_hbm.at[pl.ds(tbase, B_TAIL)], trow)
        pltpu.sync_copy(col_hbm.at[pl.ds(tbase, B_TAIL)], tcol)
        pltpu.sync_copy(w_hbm.at[pl.ds(tbase, B_TAIL)],
                        wbufs[0].at[pl.ds(0, B_TAIL)])
        pltpu.async_copy(xw_hbm.at[trow], gbufs[0].at[pl.ds(0, B_TAIL)],
                         sems[0]).wait()
        mul(0, B_TAIL)
        pltpu.sync_copy(gbufs[0].at[pl.ds(0, B_TAIL)], acc.at[tcol], add=True)

        def start(i, k, wait_scatter):
            if wait_scatter:
                pltpu.make_async_copy(gbufs[k], acc.at[cols[k]],
                                      ssems[k]).wait()
            base = wid * EPT + i * B_C
            pltpu.sync_copy(row_hbm.at[pl.ds(base, B_C)], rows[k])
            pltpu.sync_copy(col_hbm.at[pl.ds(base, B_C)], cols[k])
            pltpu.sync_copy(w_hbm.at[pl.ds(base, B_C)], wbufs[k])
            return pltpu.async_copy(xw_hbm.at[rows[k]], gbufs[k], sems[k])

        cp = start(0, 0, False)
        for i in range(B_NCH):
            k = i % 2
            nxt = None
            if i + 1 < B_NCH:
                nxt = start(i + 1, 1 - k, i + 1 >= 2)
            cp.wait()
            mul(k, B_C)
            pltpu.async_copy(gbufs[k], acc.at[cols[k]], ssems[k], add=True)
            cp = nxt
        for k in (0, 1):
            pltpu.make_async_copy(gbufs[k], acc.at[cols[k]], ssems[k]).wait()
        plsc.subcore_barrier()
        pltpu.sync_copy(acc.at[pl.ds(s * RPT, RPT)],
                        out_hbm.at[c, pl.ds(s * RPT, RPT)])

    pl.run_scoped(inner,
                  pltpu.VMEM((B_C,), jnp.int32),
                  pltpu.VMEM((B_C,), jnp.int32),
                  pltpu.VMEM((B_C, 16), jnp.float32),
                  pltpu.VMEM((B_C, HID), jnp.float32),
                  pltpu.SemaphoreType.DMA,
                  pltpu.VMEM((B_C,), jnp.int32),
                  pltpu.VMEM((B_C,), jnp.int32),
                  pltpu.VMEM((B_C, 16), jnp.float32),
                  pltpu.VMEM((B_C, HID), jnp.float32),
                  pltpu.SemaphoreType.DMA,
                  pltpu.VMEM((B_TAIL,), jnp.int32),
                  pltpu.VMEM((B_TAIL,), jnp.int32),
                  pltpu.SemaphoreType.DMA,
                  pltpu.SemaphoreType.DMA)


def _sc_gatb(xw, w, row, col, zeros128):
    return _sc_map(
        _gatb_tec,
        jax.ShapeDtypeStruct((NC, NPAD, HID), jnp.float32),
        pltpu.VMEM_SHARED((NPAD, HID), jnp.float32),
    )(xw, w, row, col, zeros128)


def _tc1_body(x, nW, nb, gW, degp, h_o, xw_o, y_o):
    h = jnp.dot(x[...], nW[...], preferred_element_type=jnp.float32) + nb[...]
    xw = jnp.dot(h, gW[...], preferred_element_type=jnp.float32)
    deg2 = degp[0][0:N] + degp[1][0:N]
    dinv = lax.rsqrt(deg2[:, 0:1] + 1.0)
    h_o[...] = h
    xw_o[...] = xw
    y_o[...] = xw * dinv


def _tc2_body(h, xw, degp, segp, gcn_b, bn0g, bn0b, gatW, af, df,
              h2_o, xw2_o, a16_o, b16_o, m_o):
    deg2 = degp[0][0:N] + degp[1][0:N]
    dinv = lax.rsqrt(deg2[:, 0:1] + 1.0)
    seg = segp[0][0:N] + segp[1][0:N]
    out = dinv * seg + (dinv * dinv) * xw[...] + gcn_b[...]
    out = jnp.maximum(out * BNS * bn0g[...] + bn0b[...], 0.0)
    h2 = h[...] + out
    xw2 = jnp.dot(h2, gatW[...], preferred_element_type=jnp.float32)
    ri = lax.broadcasted_iota(jnp.int32, (HID, HEADS), 0)
    ci = lax.broadcasted_iota(jnp.int32, (HID, HEADS), 1)
    sel = (ri // CH) == ci
    amat = jnp.where(sel, af[...], 0.0)
    dmat = jnp.where(sel, df[...], 0.0)
    asrc = jnp.dot(xw2, amat, preferred_element_type=jnp.float32)
    adst = jnp.dot(xw2, dmat, preferred_element_type=jnp.float32)
    m8 = (jnp.max(asrc, axis=0, keepdims=True)
          + jnp.max(adst, axis=0, keepdims=True))
    m8 = jnp.where(m8 > 0, m8, 0.2 * m8)
    h2_o[...] = h2
    xw2_o[...] = xw2
    a16_o[...] = jnp.concatenate([asrc, asrc], axis=1)
    b16_o[...] = jnp.concatenate([adst, adst], axis=1)
    m_o[...] = jnp.concatenate([m8, m8], axis=1)


def _tc3_body(h2, xw2, a16, b16, m16, sp, nump, gat_b, bn1g, bn1b, h3_o):
    asrc = a16[...][:, 0:HEADS]
    adst = b16[...][:, 0:HEADS]
    m8 = m16[...][:, 0:HEADS]
    x = asrc + adst
    ws = jnp.exp(jnp.where(x > 0, x, 0.2 * x) - m8)
    s = sp[0][0:N, 0:HEADS] + sp[1][0:N, 0:HEADS] + ws
    ri = lax.broadcasted_iota(jnp.int32, (HEADS, HID), 0)
    ci = lax.broadcasted_iota(jnp.int32, (HEADS, HID), 1)
    emat = jnp.where((ci // CH) == ri, 1.0, 0.0).astype(jnp.float32)
    wse = jnp.dot(ws, emat, preferred_element_type=jnp.float32)
    se = jnp.dot(s, emat, preferred_element_type=jnp.float32)
    num = nump[0][0:N] + nump[1][0:N] + wse * xw2[...]
    out = num / (se + 1e-16) + gat_b[...]
    out = jnp.maximum(out * BNS * bn1g[...] + bn1b[...], 0.0)
    h3_o[...] = h2[...] + out


def _ln(hh, g, b):
    mu = jnp.mean(hh, axis=-1, keepdims=True)
    v = jnp.mean((hh - mu) ** 2, axis=-1, keepdims=True)
    return (hh - mu) * lax.rsqrt(v + 1e-5) * g + b


def _tc4_body(h3, aggp, degp, sWl, sbl, sWr, bn2g, bn2b,
              g1W, g1b, g2W, g2b, batch2, m1W, m1b, mlg, mlb, m2W, m2b,
              l1W, l1b, llg, llb, l2W, l2b, mu_o, lv_o):
    deg2 = degp[0][0:N] + degp[1][0:N]
    degm = deg2[:, 0:1]
    agg = (aggp[0][0:N] + aggp[1][0:N]) / jnp.maximum(degm, 1.0)
    out = (jnp.dot(agg, sWl[...], preferred_element_type=jnp.float32)
           + jnp.dot(h3[...], sWr[...], preferred_element_type=jnp.float32)
           + sbl[...])
    out = jnp.maximum(out * BNS * bn2g[...] + bn2b[...], 0.0)
    h4 = h3[...] + out
    gate = jnp.dot(
        jnp.maximum(jnp.dot(h4, g1W[...], preferred_element_type=jnp.float32)
                    + g1b[...], 0.0),
        g2W[...], preferred_element_type=jnp.float32) + g2b[...]
    bsel = batch2[...] == lax.broadcasted_iota(jnp.int32, (N, NG), 1)
    gb = jnp.where(bsel, gate, -jnp.inf)
    gm = jnp.max(gb, axis=0, keepdims=True)
    gm = jnp.where(jnp.isfinite(gm), gm, 0.0)
    eg = jnp.where(bsel, jnp.exp(gate - gm), 0.0)
    sg = jnp.sum(eg, axis=0, keepdims=True)
    a = eg / (sg + 1e-16)
    hg = lax.dot_general(a, h4, (((0,), (0,)), ((), ())),
                         preferred_element_type=jnp.float32)
    mu = jnp.dot(
        jnp.maximum(_ln(jnp.dot(hg, m1W[...],
                                preferred_element_type=jnp.float32)
                        + m1b[...], mlg[...], mlb[...]), 0.0),
        m2W[...], preferred_element_type=jnp.float32) + m2b[...]
    lv = jnp.dot(
        jnp.maximum(_ln(jnp.dot(hg, l1W[...],
                                preferred_element_type=jnp.float32)
                        + l1b[...], llg[...], llb[...]), 0.0),
        l2W[...], preferred_element_type=jnp.float32) + l2b[...]
    mu_o[...] = mu
    lv_o[...] = lv


def _tc(body, outs, *args):
    return pl.pallas_call(
        body, out_shape=outs,
        compiler_params=pltpu.CompilerParams(
            vmem_limit_bytes=100 * 1024 * 1024),
    )(*args)


def kernel(x, edge_index, edge_attr, batch, node_W, node_b, edge_W, edge_b,
           gcn_W, gcn_b, gat_W, gat_asrc, gat_adst, gat_b, sage_Wl, sage_bl,
           sage_Wr, bn0_g, bn0_b, bn1_g, bn1_b, bn2_g, bn2_b, gate1_W,
           gate1_b, gate2_W, gate2_b, mu1_W, mu1_b, mu_ln_g, mu_ln_b, mu2_W,
           mu2_b, lv1_W, lv1_b, lv_ln_g, lv_ln_b, lv2_W, lv2_b):
    f32 = jnp.float32
    row = edge_index[0]
    col = edge_index[1]
    zeros16 = jnp.zeros((NPAD, 16), f32)
    zeros128 = jnp.zeros((NPAD, HID), f32)
    onepat = jnp.zeros((CNT_C, 16), f32).at[:, 0].set(1.0)
    af = gat_asrc.reshape(HID, 1)
    df = gat_adst.reshape(HID, 1)
    batch2 = batch.reshape(N, 1)

    degp = _sc_count(col, onepat, zeros16)

    h, xw, y = _tc(
        _tc1_body,
        (jax.ShapeDtypeStruct((N, HID), f32),) * 3,
        x, node_W, node_b, gcn_W, degp)

    segp = _sc_segsum(y, row, col, zeros128)

    h2, xw2, a16, b16, m16 = _tc(
        _tc2_body,
        (jax.ShapeDtypeStruct((N, HID), f32),
         jax.ShapeDtypeStruct((N, HID), f32),
         jax.ShapeDtypeStruct((N, 16), f32),
         jax.ShapeDtypeStruct((N, 16), f32),
         jax.ShapeDtypeStruct((1, 16), f32)),
        h, xw, degp, segp, gcn_b, bn0_g, bn0_b, gat_W, af, df)

    wE, sp = _sc_gata(a16, b16, row, col, m16.reshape(16), zeros16)
    nump = _sc_gatb(xw2, wE, row, col, zeros128)

    h3 = _tc(
        _tc3_body,
        jax.ShapeDtypeStruct((N, HID), f32),
        h2, xw2, a16, b16, m16, sp, nump, gat_b, bn1_g, bn1_b)

    aggp = _sc_segsum(h3, row, col, zeros128)

    mu, lv = _tc(
        _tc4_body,
        (jax.ShapeDtypeStruct((NG, 64), f32),
         jax.ShapeDtypeStruct((NG, 64), f32)),
        h3, aggp, degp, sage_Wl, sage_bl, sage_Wr, bn2_g, bn2_b,
        gate1_W, gate1_b, gate2_W, gate2_b, batch2,
        mu1_W, mu1_b, mu_ln_g, mu_ln_b, mu2_W, mu2_b,
        lv1_W, lv1_b, lv_ln_g, lv_ln_b, lv2_W, lv2_b)
    return (mu, lv)


# speedup vs baseline: 1.0060x; 1.0060x over previous
import math

import jax
import jax.numpy as jnp
from jax import lax
from jax.experimental import pallas as pl
from jax.experimental.pallas import tpu as pltpu
from jax.experimental.pallas import tpu_sc as plsc
from jax._src.pallas import mpmd

N = 10000
E = 320000
HID = 128
HEADS = 8
CH = 16
NG = 16

NC, NS = 2, 16
NW = NC * NS
EPT = E // NW
NPAD = 10240
RPT = NPAD // NS

BNS = float(1.0 / math.sqrt(1.0 + 1e-5))

_VMESH = plsc.VectorSubcoreMesh(core_axis_name="c", subcore_axis_name="s",
                                num_cores=NC, num_subcores=NS)
_SMESH = plsc.ScalarSubcoreMesh(axis_name="c", num_cores=NC)


def _sc_map(tec_fn, out_types, shared_scratch):
    def scs_fn(*refs):
        pass
    return mpmd.mpmd_map(
        [(_SMESH, scs_fn), (_VMESH, tec_fn)],
        out_types=out_types,
        scratch_types=[shared_scratch],
        compiler_params=pltpu.CompilerParams(use_tc_tiling_on_sc=False, needs_layout_passes=False, internal_scratch_in_bytes=128 * 1024),
    )


CNT_C = 200


def _count_tec(col_hbm, onepat_hbm, zeros_hbm, out_hbm, acc):
    def inner(colv, srcv, sem):
        c = lax.axis_index("c")
        s = lax.axis_index("s")
        wid = s * NC + c
        pltpu.sync_copy(zeros_hbm.at[pl.ds(s * RPT, RPT)],
                        acc.at[pl.ds(s * RPT, RPT)])
        pltpu.sync_copy(onepat_hbm, srcv)
        plsc.subcore_barrier()

        def body(i, carry):
            base = wid * EPT + i * CNT_C
            pltpu.sync_copy(col_hbm.at[pl.ds(base, CNT_C)], colv)
            pltpu.sync_copy(srcv, acc.at[colv], add=True)
            return carry

        lax.fori_loop(0, EPT // CNT_C, body, 0)
        plsc.subcore_barrier()
        pltpu.sync_copy(acc.at[pl.ds(s * RPT, RPT)],
                        out_hbm.at[c, pl.ds(s * RPT, RPT)])

    pl.run_scoped(inner,
                  pltpu.VMEM((CNT_C,), jnp.int32),
                  pltpu.VMEM((CNT_C, 16), jnp.float32),
                  pltpu.SemaphoreType.DMA)


def _sc_count(col, onepat, zeros16):
    return _sc_map(
        _count_tec,
        jax.ShapeDtypeStruct((NC, NPAD, 16), jnp.float32),
        pltpu.VMEM_SHARED((NPAD, 16), jnp.float32),
    )(col, onepat, zeros16)


SEG_C = 184
SEG_NCH = EPT // SEG_C
SEG_TAIL = EPT - SEG_NCH * SEG_C


def _seg_tec(val_hbm, row_hbm, col_hbm, zeros_hbm, out_hbm, acc):
    def inner(rowv0, colv0, buf0, sem0, rowv1, colv1, buf1, sem1, trow, tcol,
              ssem0, ssem1):
        c = lax.axis_index("c")
        s = lax.axis_index("s")
        wid = s * NC + c
        rows = (rowv0, rowv1)
        cols = (colv0, colv1)
        bufs = (buf0, buf1)
        sems = (sem0, sem1)
        ssems = (ssem0, ssem1)
        pltpu.sync_copy(zeros_hbm.at[pl.ds(s * RPT, RPT)],
                        acc.at[pl.ds(s * RPT, RPT)])
        plsc.subcore_barrier()

        tbase = wid * EPT + SEG_NCH * SEG_C
        pltpu.sync_copy(row_hbm.at[pl.ds(tbase, SEG_TAIL)], trow)
        pltpu.sync_copy(col_hbm.at[pl.ds(tbase, SEG_TAIL)], tcol)
        pltpu.async_copy(val_hbm.at[trow], bufs[0].at[pl.ds(0, SEG_TAIL)],
                         sems[0]).wait()
        pltpu.sync_copy(bufs[0].at[pl.ds(0, SEG_TAIL)], acc.at[tcol], add=True)

        def start(i, k, wait_scatter):
            if wait_scatter:
                pltpu.make_async_copy(bufs[k], acc.at[cols[k]], ssems[k]).wait()
            base = wid * EPT + i * SEG_C
            pltpu.sync_copy(row_hbm.at[pl.ds(base, SEG_C)], rows[k])
            pltpu.sync_copy(col_hbm.at[pl.ds(base, SEG_C)], cols[k])
            return pltpu.async_copy(val_hbm.at[rows[k]], bufs[k], sems[k])

        cp = start(0, 0, False)
        for i in range(SEG_NCH):
            k = i % 2
            nxt = None
            if i + 1 < SEG_NCH:
                nxt = start(i + 1, 1 - k, i + 1 >= 2)
            cp.wait()
            pltpu.async_copy(bufs[k], acc.at[cols[k]], ssems[k], add=True)
            cp = nxt
        for k in (0, 1):
            pltpu.make_async_copy(bufs[k], acc.at[cols[k]], ssems[k]).wait()
        plsc.subcore_barrier()
        pltpu.sync_copy(acc.at[pl.ds(s * RPT, RPT)],
                        out_hbm.at[c, pl.ds(s * RPT, RPT)])

    pl.run_scoped(inner,
                  pltpu.VMEM((SEG_C,), jnp.int32),
                  pltpu.VMEM((SEG_C,), jnp.int32),
                  pltpu.VMEM((SEG_C, HID), jnp.float32),
                  pltpu.SemaphoreType.DMA,
                  pltpu.VMEM((SEG_C,), jnp.int32),
                  pltpu.VMEM((SEG_C,), jnp.int32),
                  pltpu.VMEM((SEG_C, HID), jnp.float32),
                  pltpu.SemaphoreType.DMA,
                  pltpu.VMEM((SEG_TAIL,), jnp.int32),
                  pltpu.VMEM((SEG_TAIL,), jnp.int32),
                  pltpu.SemaphoreType.DMA,
                  pltpu.SemaphoreType.DMA)


def _sc_segsum(val, row, col, zeros128):
    return _sc_map(
        _seg_tec,
        jax.ShapeDtypeStruct((NC, NPAD, HID), jnp.float32),
        pltpu.VMEM_SHARED((NPAD, HID), jnp.float32),
    )(val, row, col, zeros128)


A_C = 200
A_NCH = EPT // A_C


def _gata_tec(a16_hbm, b16_hbm, row_hbm, col_hbm, m_hbm, zeros_hbm,
              w_hbm, sout_hbm, acc):
    def inner(rowv0, colv0, abuf0, bbuf0, wbuf0, sem0,
              rowv1, colv1, abuf1, bbuf1, wbuf1, sem1, mv):
        c = lax.axis_index("c")
        s = lax.axis_index("s")
        wid = s * NC + c
        rows = (rowv0, rowv1)
        cols = (colv0, colv1)
        abufs = (abuf0, abuf1)
        bbufs = (bbuf0, bbuf1)
        wbufs = (wbuf0, wbuf1)
        sems = (sem0, sem1)
        pltpu.sync_copy(zeros_hbm.at[pl.ds(s * RPT, RPT)],
                        acc.at[pl.ds(s * RPT, RPT)])
        pltpu.sync_copy(m_hbm, mv)
        plsc.subcore_barrier()
        mvec = mv[...]

        def start(i, k):
            base = wid * EPT + i * A_C
            pltpu.sync_copy(row_hbm.at[pl.ds(base, A_C)], rows[k])
            pltpu.sync_copy(col_hbm.at[pl.ds(base, A_C)], cols[k])
            cpa = pltpu.async_copy(a16_hbm.at[rows[k]], abufs[k], sems[k])
            cpb = pltpu.async_copy(b16_hbm.at[cols[k]], bbufs[k], sems[k])
            return (cpa, cpb)

        cp = start(0, 0)
        for i in range(A_NCH):
            k = i % 2
            nxt = None
            if i + 1 < A_NCH:
                nxt = start(i + 1, 1 - k)
            cp[0].wait()
            cp[1].wait()

            def ebody(e, carry2):
                x = abufs[k][e] + bbufs[k][e]
                l = jnp.where(x > 0, x, 0.2 * x)
                wbufs[k][e] = jnp.exp(l - mvec)
                return carry2

            lax.fori_loop(0, A_C, ebody, 0, unroll=4)
            base = wid * EPT + i * A_C
            pltpu.sync_copy(wbufs[k], w_hbm.at[pl.ds(base, A_C)])
            pltpu.sync_copy(wbufs[k], acc.at[cols[k]], add=True)
            cp = nxt
        plsc.subcore_barrier()
        pltpu.sync_copy(acc.at[pl.ds(s * RPT, RPT)],
                        sout_hbm.at[c, pl.ds(s * RPT, RPT)])

    pl.run_scoped(inner,
                  pltpu.VMEM((A_C,), jnp.int32),
                  pltpu.VMEM((A_C,), jnp.int32),
                  pltpu.VMEM((A_C, 16), jnp.float32),
                  pltpu.VMEM((A_C, 16), jnp.float32),
                  pltpu.VMEM((A_C, 16), jnp.float32),
                  pltpu.SemaphoreType.DMA,
                  pltpu.VMEM((A_C,), jnp.int32),
                  pltpu.VMEM((A_C,), jnp.int32),
                  pltpu.VMEM((A_C, 16), jnp.float32),
                  pltpu.VMEM((A_C, 16), jnp.float32),
                  pltpu.VMEM((A_C, 16), jnp.float32),
                  pltpu.SemaphoreType.DMA,
                  pltpu.VMEM((16,), jnp.float32))


def _sc_gata(a16, b16, row, col, m16, zeros16):
    return _sc_map(
        _gata_tec,
        (jax.ShapeDtypeStruct((E, 16), jnp.float32),
         jax.ShapeDtypeStruct((NC, NPAD, 16), jnp.float32)),
        pltpu.VMEM_SHARED((NPAD, 16), jnp.float32),
    )(a16, b16, row, col, m16, zeros16)


B_C = 144
B_NCH = EPT // B_C
B_TAIL = EPT - B_NCH * B_C


def _gatb_tec(xw_hbm, w_hbm, row_hbm, col_hbm, zeros_hbm, out_hbm, acc):
    def inner(rowv0, colv0, wbuf0, gbuf0, sem0,
              rowv1, colv1, wbuf1, gbuf1, sem1, trow, tcol, ssem0, ssem1):
        ssems = (ssem0, ssem1)
        c = lax.axis_index("c")
        s = lax.axis_index("s")
        wid = s * NC + c
        rows = (rowv0, rowv1)
        cols = (colv0, colv1)
        wbufs = (wbuf0, wbuf1)
        gbufs = (gbuf0, gbuf1)
        sems = (sem0, sem1)
        pltpu.sync_copy(zeros_hbm.at[pl.ds(s * RPT, RPT)],
                        acc.at[pl.ds(s * RPT, RPT)])
        plsc.subcore_barrier()

        def mul(k, n):
            hidx = [jnp.full((16, 1), h, jnp.int32) for h in range(HEADS)]
            dn = lax.GatherDimensionNumbers(
                offset_dims=(), collapsed_slice_dims=(0,),
                start_index_map=(0,))

            def ebody(e, carry2):
                wrow = wbufs[k][e]
                for h in range(HEADS):
                    wv = lax.gather(
                        wrow, hidx[h], dn, (1,),
                        mode=lax.GatherScatterMode.PROMISE_IN_BOUNDS)
                    g = gbufs[k][e, pl.ds(h * 16, 16)]
                    gbufs[k][e, pl.ds(h * 16, 16)] = g * wv
                return carry2
            lax.fori_loop(0, n, ebody, 0, unroll=2)

        tbase = wid * EPT + B_NCH * B_C
        pltpu.sync_copy(row_hbm.at[pl.ds(tbase, B_TAIL)], trow)
        pltpu.sync_copy(col_hbm.at[pl.ds(tbase, B_TAIL)], tcol)
        pltpu.sync_copy(w_hbm.at[pl.ds(tbase, B_TAIL)],
                        wbufs[0].at[pl.ds(0, B_TAIL)])
        pltpu.async_copy(xw_hbm.at[trow], gbufs[0].at[pl.ds(0, B_TAIL)],
                         sems[0]).wait()
        mul(0, B_TAIL)
        pltpu.sync_copy(gbufs[0].at[pl.ds(0, B_TAIL)], acc.at[tcol], add=True)

        def start(i, k, wait_scatter):
            if wait_scatter:
                pltpu.make_async_copy(gbufs[k], acc.at[cols[k]],
                                      ssems[k]).wait()
            base = wid * EPT + i * B_C
            pltpu.sync_copy(row_hbm.at[pl.ds(base, B_C)], rows[k])
            pltpu.sync_copy(col_hbm.at[pl.ds(base, B_C)], cols[k])
            pltpu.sync_copy(w_hbm.at[pl.ds(base, B_C)], wbufs[k])
            return pltpu.async_copy(xw_hbm.at[rows[k]], gbufs[k], sems[k])

        cp = start(0, 0, False)
        for i in range(B_NCH):
            k = i % 2
            nxt = None
            if i + 1 < B_NCH:
                nxt = start(i + 1, 1 - k, i + 1 >= 2)
            cp.wait()
            mul(k, B_C)
            pltpu.async_copy(gbufs[k], acc.at[cols[k]], ssems[k], add=True)
            cp = nxt
        for k in (0, 1):
            pltpu.make_async_copy(gbufs[k], acc.at[cols[k]], ssems[k]).wait()
        plsc.subcore_barrier()
        pltpu.sync_copy(acc.at[pl.ds(s * RPT, RPT)],
                        out_hbm.at[c, pl.ds(s * RPT, RPT)])

    pl.run_scoped(inner,
                  pltpu.VMEM((B_C,), jnp.int32),
                  pltpu.VMEM((B_C,), jnp.int32),
                  pltpu.VMEM((B_C, 16), jnp.float32),
                  pltpu.VMEM((B_C, HID), jnp.float32),
                  pltpu.SemaphoreType.DMA,
                  pltpu.VMEM((B_C,), jnp.int32),
                  pltpu.VMEM((B_C,), jnp.int32),
                  pltpu.VMEM((B_C, 16), jnp.float32),
                  pltpu.VMEM((B_C, HID), jnp.float32),
                  pltpu.SemaphoreType.DMA,
                  pltpu.VMEM((B_TAIL,), jnp.int32),
                  pltpu.VMEM((B_TAIL,), jnp.int32),
                  pltpu.SemaphoreType.DMA,
                  pltpu.SemaphoreType.DMA)


def _sc_gatb(xw, w, row, col, zeros128):
    return _sc_map(
        _gatb_tec,
        jax.ShapeDtypeStruct((NC, NPAD, HID), jnp.float32),
        pltpu.VMEM_SHARED((NPAD, HID), jnp.float32),
    )(xw, w, row, col, zeros128)


def _tc1_body(x, nW, nb, gW, degp, h_o, xw_o, y_o):
    h = jnp.dot(x[...], nW[...], preferred_element_type=jnp.float32) + nb[...]
    xw = jnp.dot(h, gW[...], preferred_element_type=jnp.float32)
    deg2 = degp[0][0:N] + degp[1][0:N]
    dinv = lax.rsqrt(deg2[:, 0:1] + 1.0)
    h_o[...] = h
    xw_o[...] = xw
    y_o[...] = xw * dinv


def _tc2_body(h, xw, degp, segp, gcn_b, bn0g, bn0b, gatW, af, df,
              h2_o, xw2_o, a16_o, b16_o, m_o):
    deg2 = degp[0][0:N] + degp[1][0:N]
    dinv = lax.rsqrt(deg2[:, 0:1] + 1.0)
    seg = segp[0][0:N] + segp[1][0:N]
    out = dinv * seg + (dinv * dinv) * xw[...] + gcn_b[...]
    out = jnp.maximum(out * BNS * bn0g[...] + bn0b[...], 0.0)
    h2 = h[...] + out
    xw2 = jnp.dot(h2, gatW[...], preferred_element_type=jnp.float32)
    ri = lax.broadcasted_iota(jnp.int32, (HID, HEADS), 0)
    ci = lax.broadcasted_iota(jnp.int32, (HID, HEADS), 1)
    sel = (ri // CH) == ci
    amat = jnp.where(sel, af[...], 0.0)
    dmat = jnp.where(sel, df[...], 0.0)
    asrc = jnp.dot(xw2, amat, preferred_element_type=jnp.float32)
    adst = jnp.dot(xw2, dmat, preferred_element_type=jnp.float32)
    m8 = (jnp.max(asrc, axis=0, keepdims=True)
          + jnp.max(adst, axis=0, keepdims=True))
    m8 = jnp.where(m8 > 0, m8, 0.2 * m8)
    h2_o[...] = h2
    xw2_o[...] = xw2
    a16_o[...] = jnp.concatenate([asrc, asrc], axis=1)
    b16_o[...] = jnp.concatenate([adst, adst], axis=1)
    m_o[...] = jnp.concatenate([m8, m8], axis=1)


def _tc3_body(h2, xw2, a16, b16, m16, sp, nump, gat_b, bn1g, bn1b, h3_o):
    asrc = a16[...][:, 0:HEADS]
    adst = b16[...][:, 0:HEADS]
    m8 = m16[...][:, 0:HEADS]
    x = asrc + adst
    ws = jnp.exp(jnp.where(x > 0, x, 0.2 * x) - m8)
    s = sp[0][0:N, 0:HEADS] + sp[1][0:N, 0:HEADS] + ws
    ri = lax.broadcasted_iota(jnp.int32, (HEADS, HID), 0)
    ci = lax.broadcasted_iota(jnp.int32, (HEADS, HID), 1)
    emat = jnp.where((ci // CH) == ri, 1.0, 0.0).astype(jnp.float32)
    wse = jnp.dot(ws, emat, preferred_element_type=jnp.float32)
    se = jnp.dot(s, emat, preferred_element_type=jnp.float32)
    num = nump[0][0:N] + nump[1][0:N] + wse * xw2[...]
    out = num / (se + 1e-16) + gat_b[...]
    out = jnp.maximum(out * BNS * bn1g[...] + bn1b[...], 0.0)
    h3_o[...] = h2[...] + out


def _ln(hh, g, b):
    mu = jnp.mean(hh, axis=-1, keepdims=True)
    v = jnp.mean((hh - mu) ** 2, axis=-1, keepdims=True)
    return (hh - mu) * lax.rsqrt(v + 1e-5) * g + b


def _tc4_body(h3, aggp, degp, sWl, sbl, sWr, bn2g, bn2b,
              g1W, g1b, g2W, g2b, batch2, m1W, m1b, mlg, mlb, m2W, m2b,
              l1W, l1b, llg, llb, l2W, l2b, mu_o, lv_o):
    deg2 = degp[0][0:N] + degp[1][0:N]
    degm = deg2[:, 0:1]
    agg = (aggp[0][0:N] + aggp[1][0:N]) / jnp.maximum(degm, 1.0)
    out = (jnp.dot(agg, sWl[...], preferred_element_type=jnp.float32)
           + jnp.dot(h3[...], sWr[...], preferred_element_type=jnp.float32)
           + sbl[...])
    out = jnp.maximum(out * BNS * bn2g[...] + bn2b[...], 0.0)
    h4 = h3[...] + out
    gate = jnp.dot(
        jnp.maximum(jnp.dot(h4, g1W[...], preferred_element_type=jnp.float32)
                    + g1b[...], 0.0),
        g2W[...], preferred_element_type=jnp.float32) + g2b[...]
    bsel = batch2[...] == lax.broadcasted_iota(jnp.int32, (N, NG), 1)
    gb = jnp.where(bsel, gate, -jnp.inf)
    gm = jnp.max(gb, axis=0, keepdims=True)
    gm = jnp.where(jnp.isfinite(gm), gm, 0.0)
    eg = jnp.where(bsel, jnp.exp(gate - gm), 0.0)
    sg = jnp.sum(eg, axis=0, keepdims=True)
    a = eg / (sg + 1e-16)
    hg = lax.dot_general(a, h4, (((0,), (0,)), ((), ())),
                         preferred_element_type=jnp.float32)
    mu = jnp.dot(
        jnp.maximum(_ln(jnp.dot(hg, m1W[...],
                                preferred_element_type=jnp.float32)
                        + m1b[...], mlg[...], mlb[...]), 0.0),
        m2W[...], preferred_element_type=jnp.float32) + m2b[...]
    lv = jnp.dot(
        jnp.maximum(_ln(jnp.dot(hg, l1W[...],
                                preferred_element_type=jnp.float32)
                        + l1b[...], llg[...], llb[...]), 0.0),
        l2W[...], preferred_element_type=jnp.float32) + l2b[...]
    mu_o[...] = mu
    lv_o[...] = lv


def _tc(body, outs, *args):
    return pl.pallas_call(
        body, out_shape=outs,
        compiler_params=pltpu.CompilerParams(
            vmem_limit_bytes=100 * 1024 * 1024),
    )(*args)


def kernel(x, edge_index, edge_attr, batch, node_W, node_b, edge_W, edge_b,
           gcn_W, gcn_b, gat_W, gat_asrc, gat_adst, gat_b, sage_Wl, sage_bl,
           sage_Wr, bn0_g, bn0_b, bn1_g, bn1_b, bn2_g, bn2_b, gate1_W,
           gate1_b, gate2_W, gate2_b, mu1_W, mu1_b, mu_ln_g, mu_ln_b, mu2_W,
           mu2_b, lv1_W, lv1_b, lv_ln_g, lv_ln_b, lv2_W, lv2_b):
    f32 = jnp.float32
    row = edge_index[0]
    col = edge_index[1]
    zeros16 = jnp.zeros((NPAD, 16), f32)
    zeros128 = jnp.zeros((NPAD, HID), f32)
    onepat = jnp.zeros((CNT_C, 16), f32).at[:, 0].set(1.0)
    af = gat_asrc.reshape(HID, 1)
    df = gat_adst.reshape(HID, 1)
    batch2 = batch.reshape(N, 1)

    degp = _sc_count(col, onepat, zeros16)

    h, xw, y = _tc(
        _tc1_body,
        (jax.ShapeDtypeStruct((N, HID), f32),) * 3,
        x, node_W, node_b, gcn_W, degp)

    segp = _sc_segsum(y, row, col, zeros128)

    h2, xw2, a16, b16, m16 = _tc(
        _tc2_body,
        (jax.ShapeDtypeStruct((N, HID), f32),
         jax.ShapeDtypeStruct((N, HID), f32),
         jax.ShapeDtypeStruct((N, 16), f32),
         jax.ShapeDtypeStruct((N, 16), f32),
         jax.ShapeDtypeStruct((1, 16), f32)),
        h, xw, degp, segp, gcn_b, bn0_g, bn0_b, gat_W, af, df)

    wE, sp = _sc_gata(a16, b16, row, col, m16.reshape(16), zeros16)
    nump = _sc_gatb(xw2, wE, row, col, zeros128)

    h3 = _tc(
        _tc3_body,
        jax.ShapeDtypeStruct((N, HID), f32),
        h2, xw2, a16, b16, m16, sp, nump, gat_b, bn1_g, bn1_b)

    aggp = _sc_segsum(h3, row, col, zeros128)

    mu, lv = _tc(
        _tc4_body,
        (jax.ShapeDtypeStruct((NG, 64), f32),
         jax.ShapeDtypeStruct((NG, 64), f32)),
        h3, aggp, degp, sage_Wl, sage_bl, sage_Wr, bn2_g, bn2_b,
        gate1_W, gate1_b, gate2_W, gate2_b, batch2,
        mu1_W, mu1_b, mu_ln_g, mu_ln_b, mu2_W, mu2_b,
        lv1_W, lv1_b, lv_ln_g, lv_ln_b, lv2_W, lv2_b)
    return (mu, lv)


# speedup vs baseline: 1.0731x; 1.0668x over previous
import math

import jax
import jax.numpy as jnp
from jax import lax
from jax.experimental import pallas as pl
from jax.experimental.pallas import tpu as pltpu
from jax.experimental.pallas import tpu_sc as plsc
from jax._src.pallas import mpmd

N = 10000
E = 320000
HID = 128
HEADS = 8
CH = 16
NG = 16

NC, NS = 2, 16
NW = NC * NS
EPT = E // NW
NPAD = 10240
RPT = NPAD // NS

BNS = float(1.0 / math.sqrt(1.0 + 1e-5))

_VMESH = plsc.VectorSubcoreMesh(core_axis_name="c", subcore_axis_name="s",
                                num_cores=NC, num_subcores=NS)
_SMESH = plsc.ScalarSubcoreMesh(axis_name="c", num_cores=NC)


def _sc_map(tec_fn, out_types, shared_scratch):
    def scs_fn(*refs):
        pass
    return mpmd.mpmd_map(
        [(_SMESH, scs_fn), (_VMESH, tec_fn)],
        out_types=out_types,
        scratch_types=[shared_scratch],
        compiler_params=pltpu.CompilerParams(use_tc_tiling_on_sc=False, needs_layout_passes=False, internal_scratch_in_bytes=128 * 1024),
    )


CNT_C = 1000


def _count_tec(col_hbm, onepat_hbm, zeros_hbm, out_hbm, acc):
    def inner(colv, srcv, sem):
        c = lax.axis_index("c")
        s = lax.axis_index("s")
        wid = s * NC + c
        pltpu.sync_copy(zeros_hbm.at[pl.ds(s * RPT, RPT)],
                        acc.at[pl.ds(s * RPT, RPT)])
        pltpu.sync_copy(onepat_hbm, srcv)
        plsc.subcore_barrier()

        def body(i, carry):
            base = wid * EPT + i * CNT_C
            pltpu.sync_copy(col_hbm.at[pl.ds(base, CNT_C)], colv)
            pltpu.sync_copy(srcv, acc.at[colv], add=True)
            return carry

        lax.fori_loop(0, EPT // CNT_C, body, 0)
        plsc.subcore_barrier()
        pltpu.sync_copy(acc.at[pl.ds(s * RPT, RPT)],
                        out_hbm.at[c, pl.ds(s * RPT, RPT)])

    pl.run_scoped(inner,
                  pltpu.VMEM((CNT_C,), jnp.int32),
                  pltpu.VMEM((CNT_C, 16), jnp.float32),
                  pltpu.SemaphoreType.DMA)


def _sc_count(col, onepat, zeros16):
    return _sc_map(
        _count_tec,
        jax.ShapeDtypeStruct((NC, NPAD, 16), jnp.float32),
        pltpu.VMEM_SHARED((NPAD, 16), jnp.float32),
    )(col, onepat, zeros16)


SEG_C = 184
SEG_NCH = EPT // SEG_C
SEG_TAIL = EPT - SEG_NCH * SEG_C


def _seg_tec(val_hbm, row_hbm, col_hbm, zeros_hbm, out_hbm, acc):
    def inner(rowv0, colv0, buf0, sem0, rowv1, colv1, buf1, sem1, trow, tcol,
              ssem0, ssem1):
        c = lax.axis_index("c")
        s = lax.axis_index("s")
        wid = s * NC + c
        rows = (rowv0, rowv1)
        cols = (colv0, colv1)
        bufs = (buf0, buf1)
        sems = (sem0, sem1)
        ssems = (ssem0, ssem1)
        pltpu.sync_copy(zeros_hbm.at[pl.ds(s * RPT, RPT)],
                        acc.at[pl.ds(s * RPT, RPT)])
        plsc.subcore_barrier()

        tbase = wid * EPT + SEG_NCH * SEG_C
        pltpu.sync_copy(row_hbm.at[pl.ds(tbase, SEG_TAIL)], trow)
        pltpu.sync_copy(col_hbm.at[pl.ds(tbase, SEG_TAIL)], tcol)
        pltpu.async_copy(val_hbm.at[trow], bufs[0].at[pl.ds(0, SEG_TAIL)],
                         sems[0]).wait()
        pltpu.sync_copy(bufs[0].at[pl.ds(0, SEG_TAIL)], acc.at[tcol], add=True)

        def start(i, k, wait_scatter):
            if wait_scatter:
                pltpu.make_async_copy(bufs[k], acc.at[cols[k]], ssems[k]).wait()
            base = wid * EPT + i * SEG_C
            pltpu.sync_copy(row_hbm.at[pl.ds(base, SEG_C)], rows[k])
            pltpu.sync_copy(col_hbm.at[pl.ds(base, SEG_C)], cols[k])
            return pltpu.async_copy(val_hbm.at[rows[k]], bufs[k], sems[k])

        cp = start(0, 0, False)
        for i in range(SEG_NCH):
            k = i % 2
            nxt = None
            if i + 1 < SEG_NCH:
                nxt = start(i + 1, 1 - k, i + 1 >= 2)
            cp.wait()
            pltpu.async_copy(bufs[k], acc.at[cols[k]], ssems[k], add=True)
            cp = nxt
        for k in (0, 1):
            pltpu.make_async_copy(bufs[k], acc.at[cols[k]], ssems[k]).wait()
        plsc.subcore_barrier()
        pltpu.sync_copy(acc.at[pl.ds(s * RPT, RPT)],
                        out_hbm.at[c, pl.ds(s * RPT, RPT)])

    pl.run_scoped(inner,
                  pltpu.VMEM((SEG_C,), jnp.int32),
                  pltpu.VMEM((SEG_C,), jnp.int32),
                  pltpu.VMEM((SEG_C, HID), jnp.float32),
                  pltpu.SemaphoreType.DMA,
                  pltpu.VMEM((SEG_C,), jnp.int32),
                  pltpu.VMEM((SEG_C,), jnp.int32),
                  pltpu.VMEM((SEG_C, HID), jnp.float32),
                  pltpu.SemaphoreType.DMA,
                  pltpu.VMEM((SEG_TAIL,), jnp.int32),
                  pltpu.VMEM((SEG_TAIL,), jnp.int32),
                  pltpu.SemaphoreType.DMA,
                  pltpu.SemaphoreType.DMA)


def _sc_segsum(val, row, col, zeros128):
    return _sc_map(
        _seg_tec,
        jax.ShapeDtypeStruct((NC, NPAD, HID), jnp.float32),
        pltpu.VMEM_SHARED((NPAD, HID), jnp.float32),
    )(val, row, col, zeros128)


A_C = 400
A_NCH = EPT // A_C


def _gata_tec(a16_hbm, b16_hbm, row_hbm, col_hbm, m_hbm, zeros_hbm,
              w_hbm, sout_hbm, acc):
    def inner(rowv0, colv0, abuf0, bbuf0, wbuf0, sem0,
              rowv1, colv1, abuf1, bbuf1, wbuf1, sem1, mv):
        c = lax.axis_index("c")
        s = lax.axis_index("s")
        wid = s * NC + c
        rows = (rowv0, rowv1)
        cols = (colv0, colv1)
        abufs = (abuf0, abuf1)
        bbufs = (bbuf0, bbuf1)
        wbufs = (wbuf0, wbuf1)
        sems = (sem0, sem1)
        pltpu.sync_copy(zeros_hbm.at[pl.ds(s * RPT, RPT)],
                        acc.at[pl.ds(s * RPT, RPT)])
        pltpu.sync_copy(m_hbm, mv)
        plsc.subcore_barrier()
        mvec = mv[...]

        def start(i, k):
            base = wid * EPT + i * A_C
            pltpu.sync_copy(row_hbm.at[pl.ds(base, A_C)], rows[k])
            pltpu.sync_copy(col_hbm.at[pl.ds(base, A_C)], cols[k])
            cpa = pltpu.async_copy(a16_hbm.at[rows[k]], abufs[k], sems[k])
            cpb = pltpu.async_copy(b16_hbm.at[cols[k]], bbufs[k], sems[k])
            return (cpa, cpb)

        cp = start(0, 0)
        for i in range(A_NCH):
            k = i % 2
            nxt = None
            if i + 1 < A_NCH:
                nxt = start(i + 1, 1 - k)
            cp[0].wait()
            cp[1].wait()

            def ebody(e, carry2):
                x = abufs[k][e] + bbufs[k][e]
                l = jnp.where(x > 0, x, 0.2 * x)
                wbufs[k][e] = jnp.exp(l - mvec)
                return carry2

            lax.fori_loop(0, A_C, ebody, 0, unroll=4)
            base = wid * EPT + i * A_C
            pltpu.sync_copy(wbufs[k], w_hbm.at[pl.ds(base, A_C)])
            pltpu.sync_copy(wbufs[k], acc.at[cols[k]], add=True)
            cp = nxt
        plsc.subcore_barrier()
        pltpu.sync_copy(acc.at[pl.ds(s * RPT, RPT)],
                        sout_hbm.at[c, pl.ds(s * RPT, RPT)])

    pl.run_scoped(inner,
                  pltpu.VMEM((A_C,), jnp.int32),
                  pltpu.VMEM((A_C,), jnp.int32),
                  pltpu.VMEM((A_C, 16), jnp.float32),
                  pltpu.VMEM((A_C, 16), jnp.float32),
                  pltpu.VMEM((A_C, 16), jnp.float32),
                  pltpu.SemaphoreType.DMA,
                  pltpu.VMEM((A_C,), jnp.int32),
                  pltpu.VMEM((A_C,), jnp.int32),
                  pltpu.VMEM((A_C, 16), jnp.float32),
                  pltpu.VMEM((A_C, 16), jnp.float32),
                  pltpu.VMEM((A_C, 16), jnp.float32),
                  pltpu.SemaphoreType.DMA,
                  pltpu.VMEM((16,), jnp.float32))


def _sc_gata(a16, b16, row, col, m16, zeros16):
    return _sc_map(
        _gata_tec,
        (jax.ShapeDtypeStruct((E, 16), jnp.float32),
         jax.ShapeDtypeStruct((NC, NPAD, 16), jnp.float32)),
        pltpu.VMEM_SHARED((NPAD, 16), jnp.float32),
    )(a16, b16, row, col, m16, zeros16)


B_C = 144
B_NCH = EPT // B_C
B_TAIL = EPT - B_NCH * B_C


def _gatb_tec(xw_hbm, w_hbm, row_hbm, col_hbm, zeros_hbm, out_hbm, acc):
    def inner(rowv0, colv0, wbuf0, gbuf0, sem0,
              rowv1, colv1, wbuf1, gbuf1, sem1, trow, tcol, ssem0, ssem1):
        ssems = (ssem0, ssem1)
        c = lax.axis_index("c")
        s = lax.axis_index("s")
        wid = s * NC + c
        rows = (rowv0, rowv1)
        cols = (colv0, colv1)
        wbufs = (wbuf0, wbuf1)
        gbufs = (gbuf0, gbuf1)
        sems = (sem0, sem1)
        pltpu.sync_copy(zeros_hbm.at[pl.ds(s * RPT, RPT)],
                        acc.at[pl.ds(s * RPT, RPT)])
        plsc.subcore_barrier()

        def mul(k, n):
            hidx = [jnp.full((16, 1), h, jnp.int32) for h in range(HEADS)]
            dn = lax.GatherDimensionNumbers(
                offset_dims=(), collapsed_slice_dims=(0,),
                start_index_map=(0,))

            def ebody(e, carry2):
                wrow = wbufs[k][e]
                for h in range(HEADS):
                    wv = lax.gather(
                        wrow, hidx[h], dn, (1,),
                        mode=lax.GatherScatterMode.PROMISE_IN_BOUNDS)
                    g = gbufs[k][e, pl.ds(h * 16, 16)]
                    gbufs[k][e, pl.ds(h * 16, 16)] = g * wv
                return carry2
            lax.fori_loop(0, n, ebody, 0, unroll=2)

        tbase = wid * EPT + B_NCH * B_C
        pltpu.sync_copy(row_hbm.at[pl.ds(tbase, B_TAIL)], trow)
        pltpu.sync_copy(col_hbm.at[pl.ds(tbase, B_TAIL)], tcol)
        pltpu.sync_copy(w_hbm.at[pl.ds(tbase, B_TAIL)],
                        wbufs[0].at[pl.ds(0, B_TAIL)])
        pltpu.async_copy(xw_hbm.at[trow], gbufs[0].at[pl.ds(0, B_TAIL)],
                         sems[0]).wait()
        mul(0, B_TAIL)
        pltpu.sync_copy(gbufs[0].at[pl.ds(0, B_TAIL)], acc.at[tcol], add=True)

        def start(i, k, wait_scatter):
            if wait_scatter:
                pltpu.make_async_copy(gbufs[k], acc.at[cols[k]],
                                      ssems[k]).wait()
            base = wid * EPT + i * B_C
            pltpu.sync_copy(row_hbm.at[pl.ds(base, B_C)], rows[k])
            pltpu.sync_copy(col_hbm.at[pl.ds(base, B_C)], cols[k])
            pltpu.sync_copy(w_hbm.at[pl.ds(base, B_C)], wbufs[k])
            return pltpu.async_copy(xw_hbm.at[rows[k]], gbufs[k], sems[k])

        cp = start(0, 0, False)
        for i in range(B_NCH):
            k = i % 2
            nxt = None
            if i + 1 < B_NCH:
                nxt = start(i + 1, 1 - k, i + 1 >= 2)
            cp.wait()
            mul(k, B_C)
            pltpu.async_copy(gbufs[k], acc.at[cols[k]], ssems[k], add=True)
            cp = nxt
        for k in (0, 1):
            pltpu.make_async_copy(gbufs[k], acc.at[cols[k]], ssems[k]).wait()
        plsc.subcore_barrier()
        pltpu.sync_copy(acc.at[pl.ds(s * RPT, RPT)],
                        out_hbm.at[c, pl.ds(s * RPT, RPT)])

    pl.run_scoped(inner,
                  pltpu.VMEM((B_C,), jnp.int32),
                  pltpu.VMEM((B_C,), jnp.int32),
                  pltpu.VMEM((B_C, 16), jnp.float32),
                  pltpu.VMEM((B_C, HID), jnp.float32),
                  pltpu.SemaphoreType.DMA,
                  pltpu.VMEM((B_C,), jnp.int32),
                  pltpu.VMEM((B_C,), jnp.int32),
                  pltpu.VMEM((B_C, 16), jnp.float32),
                  pltpu.VMEM((B_C, HID), jnp.float32),
                  pltpu.SemaphoreType.DMA,
                  pltpu.VMEM((B_TAIL,), jnp.int32),
                  pltpu.VMEM((B_TAIL,), jnp.int32),
                  pltpu.SemaphoreType.DMA,
                  pltpu.SemaphoreType.DMA)


def _sc_gatb(xw, w, row, col, zeros128):
    return _sc_map(
        _gatb_tec,
        jax.ShapeDtypeStruct((NC, NPAD, HID), jnp.float32),
        pltpu.VMEM_SHARED((NPAD, HID), jnp.float32),
    )(xw, w, row, col, zeros128)


def _tc1_body(x, nW, nb, gW, degp, h_o, xw_o, y_o):
    h = jnp.dot(x[...], nW[...], preferred_element_type=jnp.float32) + nb[...]
    xw = jnp.dot(h, gW[...], preferred_element_type=jnp.float32)
    deg2 = degp[0][0:N] + degp[1][0:N]
    dinv = lax.rsqrt(deg2[:, 0:1] + 1.0)
    h_o[...] = h
    xw_o[...] = xw
    y_o[...] = xw * dinv


def _tc2_body(h, xw, degp, segp, gcn_b, bn0g, bn0b, gatW, af, df,
              h2_o, xw2_o, a16_o, b16_o, m_o):
    deg2 = degp[0][0:N] + degp[1][0:N]
    dinv = lax.rsqrt(deg2[:, 0:1] + 1.0)
    seg = segp[0][0:N] + segp[1][0:N]
    out = dinv * seg + (dinv * dinv) * xw[...] + gcn_b[...]
    out = jnp.maximum(out * BNS * bn0g[...] + bn0b[...], 0.0)
    h2 = h[...] + out
    xw2 = jnp.dot(h2, gatW[...], preferred_element_type=jnp.float32)
    ri = lax.broadcasted_iota(jnp.int32, (HID, HEADS), 0)
    ci = lax.broadcasted_iota(jnp.int32, (HID, HEADS), 1)
    sel = (ri // CH) == ci
    amat = jnp.where(sel, af[...], 0.0)
    dmat = jnp.where(sel, df[...], 0.0)
    asrc = jnp.dot(xw2, amat, preferred_element_type=jnp.float32)
    adst = jnp.dot(xw2, dmat, preferred_element_type=jnp.float32)
    m8 = (jnp.max(asrc, axis=0, keepdims=True)
          + jnp.max(adst, axis=0, keepdims=True))
    m8 = jnp.where(m8 > 0, m8, 0.2 * m8)
    h2_o[...] = h2
    xw2_o[...] = xw2
    a16_o[...] = jnp.concatenate([asrc, asrc], axis=1)
    b16_o[...] = jnp.concatenate([adst, adst], axis=1)
    m_o[...] = jnp.concatenate([m8, m8], axis=1)


def _tc3_body(h2, xw2, a16, b16, m16, sp, nump, gat_b, bn1g, bn1b, h3_o):
    asrc = a16[...][:, 0:HEADS]
    adst = b16[...][:, 0:HEADS]
    m8 = m16[...][:, 0:HEADS]
    x = asrc + adst
    ws = jnp.exp(jnp.where(x > 0, x, 0.2 * x) - m8)
    s = sp[0][0:N, 0:HEADS] + sp[1][0:N, 0:HEADS] + ws
    ri = lax.broadcasted_iota(jnp.int32, (HEADS, HID), 0)
    ci = lax.broadcasted_iota(jnp.int32, (HEADS, HID), 1)
    emat = jnp.where((ci // CH) == ri, 1.0, 0.0).astype(jnp.float32)
    wse = jnp.dot(ws, emat, preferred_element_type=jnp.float32)
    se = jnp.dot(s, emat, preferred_element_type=jnp.float32)
    num = nump[0][0:N] + nump[1][0:N] + wse * xw2[...]
    out = num / (se + 1e-16) + gat_b[...]
    out = jnp.maximum(out * BNS * bn1g[...] + bn1b[...], 0.0)
    h3_o[...] = h2[...] + out


def _ln(hh, g, b):
    mu = jnp.mean(hh, axis=-1, keepdims=True)
    v = jnp.mean((hh - mu) ** 2, axis=-1, keepdims=True)
    return (hh - mu) * lax.rsqrt(v + 1e-5) * g + b


def _tc4_body(h3, aggp, degp, sWl, sbl, sWr, bn2g, bn2b,
              g1W, g1b, g2W, g2b, batch2, m1W, m1b, mlg, mlb, m2W, m2b,
              l1W, l1b, llg, llb, l2W, l2b, mu_o, lv_o):
    deg2 = degp[0][0:N] + degp[1][0:N]
    degm = deg2[:, 0:1]
    agg = (aggp[0][0:N] + aggp[1][0:N]) / jnp.maximum(degm, 1.0)
    out = (jnp.dot(agg, sWl[...], preferred_element_type=jnp.float32)
           + jnp.dot(h3[...], sWr[...], preferred_element_type=jnp.float32)
           + sbl[...])
    out = jnp.maximum(out * BNS * bn2g[...] + bn2b[...], 0.0)
    h4 = h3[...] + out
    gate = jnp.dot(
        jnp.maximum(jnp.dot(h4, g1W[...], preferred_element_type=jnp.float32)
                    + g1b[...], 0.0),
        g2W[...], preferred_element_type=jnp.float32) + g2b[...]
    bsel = batch2[...] == lax.broadcasted_iota(jnp.int32, (N, NG), 1)
    gb = jnp.where(bsel, gate, -jnp.inf)
    gm = jnp.max(gb, axis=0, keepdims=True)
    gm = jnp.where(jnp.isfinite(gm), gm, 0.0)
    eg = jnp.where(bsel, jnp.exp(gate - gm), 0.0)
    sg = jnp.sum(eg, axis=0, keepdims=True)
    a = eg / (sg + 1e-16)
    hg = lax.dot_general(a, h4, (((0,), (0,)), ((), ())),
                         preferred_element_type=jnp.float32)
    mu = jnp.dot(
        jnp.maximum(_ln(jnp.dot(hg, m1W[...],
                                preferred_element_type=jnp.float32)
                        + m1b[...], mlg[...], mlb[...]), 0.0),
        m2W[...], preferred_element_type=jnp.float32) + m2b[...]
    lv = jnp.dot(
        jnp.maximum(_ln(jnp.dot(hg, l1W[...],
                                preferred_element_type=jnp.float32)
                        + l1b[...], llg[...], llb[...]), 0.0),
        l2W[...], preferred_element_type=jnp.float32) + l2b[...]
    mu_o[...] = mu
    lv_o[...] = lv


def _tc(body, outs, *args):
    return pl.pallas_call(
        body, out_shape=outs,
        compiler_params=pltpu.CompilerParams(
            vmem_limit_bytes=100 * 1024 * 1024),
    )(*args)


def kernel(x, edge_index, edge_attr, batch, node_W, node_b, edge_W, edge_b,
           gcn_W, gcn_b, gat_W, gat_asrc, gat_adst, gat_b, sage_Wl, sage_bl,
           sage_Wr, bn0_g, bn0_b, bn1_g, bn1_b, bn2_g, bn2_b, gate1_W,
           gate1_b, gate2_W, gate2_b, mu1_W, mu1_b, mu_ln_g, mu_ln_b, mu2_W,
           mu2_b, lv1_W, lv1_b, lv_ln_g, lv_ln_b, lv2_W, lv2_b):
    f32 = jnp.float32
    row = edge_index[0]
    col = edge_index[1]
    zeros16 = jnp.zeros((NPAD, 16), f32)
    zeros128 = jnp.zeros((NPAD, HID), f32)
    onepat = jnp.zeros((CNT_C, 16), f32).at[:, 0].set(1.0)
    af = gat_asrc.reshape(HID, 1)
    df = gat_adst.reshape(HID, 1)
    batch2 = batch.reshape(N, 1)

    degp = _sc_count(col, onepat, zeros16)

    h, xw, y = _tc(
        _tc1_body,
        (jax.ShapeDtypeStruct((N, HID), f32),) * 3,
        x, node_W, node_b, gcn_W, degp)

    segp = _sc_segsum(y, row, col, zeros128)

    h2, xw2, a16, b16, m16 = _tc(
        _tc2_body,
        (jax.ShapeDtypeStruct((N, HID), f32),
         jax.ShapeDtypeStruct((N, HID), f32),
         jax.ShapeDtypeStruct((N, 16), f32),
         jax.ShapeDtypeStruct((N, 16), f32),
         jax.ShapeDtypeStruct((1, 16), f32)),
        h, xw, degp, segp, gcn_b, bn0_g, bn0_b, gat_W, af, df)

    wE, sp = _sc_gata(a16, b16, row, col, m16.reshape(16), zeros16)
    nump = _sc_gatb(xw2, wE, row, col, zeros128)

    h3 = _tc(
        _tc3_body,
        jax.ShapeDtypeStruct((N, HID), f32),
        h2, xw2, a16, b16, m16, sp, nump, gat_b, bn1_g, bn1_b)

    aggp = _sc_segsum(h3, row, col, zeros128)

    mu, lv = _tc(
        _tc4_body,
        (jax.ShapeDtypeStruct((NG, 64), f32),
         jax.ShapeDtypeStruct((NG, 64), f32)),
        h3, aggp, degp, sage_Wl, sage_bl, sage_Wr, bn2_g, bn2_b,
        gate1_W, gate1_b, gate2_W, gate2_b, batch2,
        mu1_W, mu1_b, mu_ln_g, mu_ln_b, mu2_W, mu2_b,
        lv1_W, lv1_b, lv_ln_g, lv_ln_b, lv2_W, lv2_b)
    return (mu, lv)
